# baseline (speedup 1.0000x reference)
import functools

import jax
import jax.numpy as jnp
from jax import lax
from jax.experimental import pallas as pl
from jax.experimental.pallas import tpu as pltpu
from jax.experimental.pallas import tpu_sc as plsc

_L = 16
_C = 80


def _sc_body(epw, num_cores, h_src, h_dst, s_idx, d_idx, out,
             sidx_v, didx_v, srows, drows, mat_v, scores_v, sems, semd):
    wid = lax.axis_index("s") * num_cores + lax.axis_index("c")
    nchk = epw // _C
    wbase = wid * epw

    pltpu.sync_copy(s_idx.at[pl.ds(wbase, epw)], sidx_v)
    pltpu.sync_copy(d_idx.at[pl.ds(wbase, epw)], didx_v)

    def start(chunk, b):
        pltpu.async_copy(h_src.at[sidx_v.at[pl.ds(chunk * _C, _C)]],
                         srows[b], sems[b])
        pltpu.async_copy(h_dst.at[didx_v.at[pl.ds(chunk * _C, _C)]],
                         drows[b], semd[b])

    start(0, 0)
    start(1, 1)

    lanes = lax.iota(jnp.int32, _L)

    def compute(chunk, b):
        sr, dr = srows[b], drows[b]

        def group_body(g, _):
            for e16 in range(_L):
                e = g * _L + e16
                acc = None
                for k in range(4):
                    sw = plsc.bitcast(sr[e, pl.ds(k * _L, _L)], jnp.bfloat16)
                    dw = plsc.bitcast(dr[e, pl.ds(k * _L, _L)], jnp.bfloat16)
                    sa, sb = plsc.unpack(sw, format=plsc.PackFormat.INTERLEAVED)
                    da, db = plsc.unpack(dw, format=plsc.PackFormat.INTERLEAVED)
                    term = sa * da + sb * db
                    acc = term if acc is None else acc + term
                mat_v[pl.ds(e16 * _L, _L)] = acc
            tot = plsc.load_gather(mat_v, [lanes * _L])
            for j in range(1, _L):
                tot = tot + plsc.load_gather(mat_v, [lanes * _L + j])
            scores_v[pl.ds(chunk * _C + g * _L, _L)] = tot
            return 0

        lax.fori_loop(0, _C // _L, group_body, 0)

    def pair_body(i2, _):
        for b in range(2):
            i = i2 * 2 + b

            @pl.when(i < nchk)
            def _():
                pltpu.make_async_copy(
                    h_src.at[sidx_v.at[pl.ds(i * _C, _C)]], srows[b], sems[b]
                ).wait()
                pltpu.make_async_copy(
                    h_dst.at[didx_v.at[pl.ds(i * _C, _C)]], drows[b], semd[b]
                ).wait()
                compute(i, b)

                @pl.when(i + 2 < nchk)
                def _():
                    start(i + 2, b)
        return 0

    lax.fori_loop(0, (nchk + 1) // 2, pair_body, 0)
    pltpu.sync_copy(scores_v, out.at[pl.ds(wbase, epw)])


def kernel(h_src, h_dst, edge_index):
    n_nodes, d_feat = h_src.shape
    n_edges = edge_index.shape[1]
    assert d_feat == 128

    s_idx = edge_index[0].astype(jnp.int32)
    d_idx = edge_index[1].astype(jnp.int32)
    h_src = lax.bitcast_convert_type(
        h_src.astype(jnp.bfloat16).reshape(n_nodes, d_feat // 2, 2), jnp.int32)
    h_dst = lax.bitcast_convert_type(
        h_dst.astype(jnp.bfloat16).reshape(n_nodes, d_feat // 2, 2), jnp.int32)

    mesh = plsc.VectorSubcoreMesh(core_axis_name="c", subcore_axis_name="s")
    num_cores = mesh.num_cores
    nw = num_cores * mesh.num_subcores
    assert n_edges % (nw * _C) == 0
    epw = n_edges // nw

    sc_fn = pl.kernel(
        functools.partial(_sc_body, epw, num_cores),
        out_type=jax.ShapeDtypeStruct((n_edges,), jnp.float32),
        mesh=mesh,
        scratch_types=[
            pltpu.VMEM((epw,), jnp.int32),
            pltpu.VMEM((epw,), jnp.int32),
            [pltpu.VMEM((_C, 64), jnp.int32) for _ in range(2)],
            [pltpu.VMEM((_C, 64), jnp.int32) for _ in range(2)],
            pltpu.VMEM((_L * _L,), jnp.float32),
            pltpu.VMEM((epw,), jnp.float32),
            [pltpu.SemaphoreType.DMA for _ in range(2)],
            [pltpu.SemaphoreType.DMA for _ in range(2)],
        ],
        compiler_params=pltpu.CompilerParams(
            needs_layout_passes=False, use_tc_tiling_on_sc=False),
    )
    scores = sc_fn(h_src, h_dst, s_idx, d_idx)
    return scores.reshape(n_edges, 1)


# speedup vs baseline: 1.2439x; 1.2439x over previous
import functools

import jax
import jax.numpy as jnp
from jax import lax
from jax.experimental import pallas as pl
from jax.experimental.pallas import tpu as pltpu
from jax.experimental.pallas import tpu_sc as plsc

_L = 16
_C = 80


def _sc_body(epw, num_cores, h_src, h_dst, s_idx, d_idx, out,
             sidx_v, didx_v, srows, drows, mat_v, scores_v, sems, semd):
    wid = lax.axis_index("s") * num_cores + lax.axis_index("c")
    nchk = epw // _C
    wbase = wid * epw

    pltpu.sync_copy(s_idx.at[pl.ds(wbase, epw)], sidx_v)
    pltpu.sync_copy(d_idx.at[pl.ds(wbase, epw)], didx_v)

    def start(chunk, b):
        pltpu.async_copy(h_src.at[sidx_v.at[pl.ds(chunk * _C, _C)]],
                         srows[b], sems[b])
        pltpu.async_copy(h_dst.at[didx_v.at[pl.ds(chunk * _C, _C)]],
                         drows[b], semd[b])

    start(0, 0)
    start(1, 1)

    lanes = lax.iota(jnp.int32, _L)

    def compute(chunk, b):
        sr, dr = srows[b], drows[b]

        def group_body(g, _):
            for e16 in range(_L):
                e = g * _L + e16
                acc = sr[e, pl.ds(0, _L)] * dr[e, pl.ds(0, _L)]
                for k in range(1, 8):
                    acc = acc + (sr[e, pl.ds(k * _L, _L)]
                                 * dr[e, pl.ds(k * _L, _L)])
                mat_v[pl.ds(e16 * _L, _L)] = acc
            tot = plsc.load_gather(mat_v, [lanes * _L])
            for j in range(1, _L):
                tot = tot + plsc.load_gather(mat_v, [lanes * _L + j])
            scores_v[pl.ds(chunk * _C + g * _L, _L)] = tot
            return 0

        lax.fori_loop(0, _C // _L, group_body, 0)

    def pair_body(i2, _):
        for b in range(2):
            i = i2 * 2 + b

            @pl.when(i < nchk)
            def _():
                pltpu.make_async_copy(
                    h_src.at[sidx_v.at[pl.ds(i * _C, _C)]], srows[b], sems[b]
                ).wait()
                pltpu.make_async_copy(
                    h_dst.at[didx_v.at[pl.ds(i * _C, _C)]], drows[b], semd[b]
                ).wait()
                compute(i, b)

                @pl.when(i + 2 < nchk)
                def _():
                    start(i + 2, b)
        return 0

    lax.fori_loop(0, (nchk + 1) // 2, pair_body, 0)
    pltpu.sync_copy(scores_v, out.at[pl.ds(wbase, epw)])


def kernel(h_src, h_dst, edge_index):
    n_nodes, d_feat = h_src.shape
    n_edges = edge_index.shape[1]
    assert d_feat == 128

    s_idx = edge_index[0].astype(jnp.int32)
    d_idx = edge_index[1].astype(jnp.int32)

    mesh = plsc.VectorSubcoreMesh(core_axis_name="c", subcore_axis_name="s")
    num_cores = mesh.num_cores
    nw = num_cores * mesh.num_subcores
    assert n_edges % (nw * _C) == 0
    epw = n_edges // nw

    sc_fn = pl.kernel(
        functools.partial(_sc_body, epw, num_cores),
        out_type=jax.ShapeDtypeStruct((n_edges,), jnp.float32),
        mesh=mesh,
        scratch_types=[
            pltpu.VMEM((epw,), jnp.int32),
            pltpu.VMEM((epw,), jnp.int32),
            [pltpu.VMEM((_C, 128), jnp.float32) for _ in range(2)],
            [pltpu.VMEM((_C, 128), jnp.float32) for _ in range(2)],
            pltpu.VMEM((_L * _L,), jnp.float32),
            pltpu.VMEM((epw,), jnp.float32),
            [pltpu.SemaphoreType.DMA for _ in range(2)],
            [pltpu.SemaphoreType.DMA for _ in range(2)],
        ],
        compiler_params=pltpu.CompilerParams(needs_layout_passes=False),
    )
    scores = sc_fn(h_src, h_dst, s_idx, d_idx)
    return scores.reshape(n_edges, 1)
